# baseline (speedup 1.0000x reference)
import functools

import jax
import jax.numpy as jnp
from jax import lax
from jax.experimental import pallas as pl
from jax.experimental.pallas import tpu as pltpu
from jax.experimental.pallas import tpu_sc as plsc

VOCAB_ROWS = 32
EMBED_DIM = 64
BATCH = 4096
SEQ = 200
TOTAL = BATCH * SEQ

_info = plsc.get_sparse_core_info()
_NC = _info.num_cores
_NS = _info.num_subcores
_NW = _NC * _NS
_L = _info.num_lanes
PER_W = TOTAL // _NW
BUF_ROWS = 128
NSTEP = PER_W // BUF_ROWS
NGROUP = BUF_ROWS // _L
NBUF = 4


def _make_kernel():
    mesh = plsc.VectorSubcoreMesh(core_axis_name="c", subcore_axis_name="s")

    @functools.partial(
        pl.kernel,
        mesh=mesh,
        out_type=jax.ShapeDtypeStruct((TOTAL * EMBED_DIM,), jnp.float32),
        compiler_params=pltpu.CompilerParams(
            use_tc_tiling_on_sc=False, needs_layout_passes=False),
        scratch_types=[
            pltpu.VMEM((PER_W,), jnp.int32),
            pltpu.VMEM((NBUF, BUF_ROWS * EMBED_DIM), jnp.float32),
            pltpu.VMEM((VOCAB_ROWS * EMBED_DIM,), jnp.float32),
        ]
        + [pltpu.SemaphoreType.DMA] * NBUF,
    )
    def k(idx_hbm, table_hbm, out_hbm, idx_v, rows, table_v,
          o0, o1, o2, o3):
        osem = [o0, o1, o2, o3]
        wid = lax.axis_index("s") * _NC + lax.axis_index("c")
        base = wid * PER_W
        obase = base * EMBED_DIM

        pltpu.sync_copy(table_hbm, table_v)
        pltpu.sync_copy(idx_hbm.at[pl.ds(base, PER_W)], idx_v)

        ovec = jax.lax.iota(jnp.int32, _L) * EMBED_DIM

        def compute(s, b):
            buf = rows.at[b]

            def group(g, carry):
                idx_vec = idx_v[pl.ds(s * BUF_ROWS + g * _L, _L)]
                gvec = idx_vec << 6
                dvec = ovec + g * (_L * EMBED_DIM)
                for c in range(EMBED_DIM):
                    v = plsc.load_gather(table_v, [gvec + c])
                    plsc.store_scatter(buf, [dvec + c], v)
                return carry

            lax.fori_loop(0, NGROUP, group, 0)

        def write(s, b, start):
            cp = pltpu.make_async_copy(
                rows.at[b],
                out_hbm.at[pl.ds(obase + s * (BUF_ROWS * EMBED_DIM),
                                 BUF_ROWS * EMBED_DIM)],
                osem[b],
            )
            cp.start() if start else cp.wait()

        for b in range(NBUF):
            compute(b, b)
            write(b, b, True)

        def body(i, carry):
            sbase = i * NBUF
            for b in range(NBUF):
                s = sbase + b
                write(s - NBUF, b, False)
                compute(s, b)
                write(s, b, True)
            return carry

        lax.fori_loop(1, NSTEP // NBUF, body, 0)

        for b in range(NBUF):
            write(NSTEP - NBUF + b, b, False)

    return k


_sc_gather = _make_kernel()


def kernel(inputs, embedding_table):
    idx = inputs.reshape(TOTAL)
    table = embedding_table.reshape(VOCAB_ROWS * EMBED_DIM)
    out = _sc_gather(idx, table)
    return out.reshape(BATCH, SEQ, EMBED_DIM)


# speedup vs baseline: 2.6888x; 2.6888x over previous
import functools

import jax
import jax.numpy as jnp
from jax import lax
from jax.experimental import pallas as pl
from jax.experimental.pallas import tpu as pltpu
from jax.experimental.pallas import tpu_sc as plsc

VOCAB_ROWS = 32
EMBED_DIM = 64
BATCH = 4096
SEQ = 200
TOTAL = BATCH * SEQ

_info = plsc.get_sparse_core_info()
_NC = _info.num_cores
_NS = _info.num_subcores
_NW = _NC * _NS
_L = _info.num_lanes
PER_W = TOTAL // _NW
BUF_ROWS = 128
NSTEP = PER_W // BUF_ROWS
NGROUP = BUF_ROWS // _L
NBUF = 4


def _make_kernel():
    mesh = plsc.VectorSubcoreMesh(core_axis_name="c", subcore_axis_name="s")

    @functools.partial(
        pl.kernel,
        mesh=mesh,
        out_type=jax.ShapeDtypeStruct((TOTAL * EMBED_DIM,), jnp.float32),
        compiler_params=pltpu.CompilerParams(
            use_tc_tiling_on_sc=False, needs_layout_passes=False),
        scratch_types=[
            pltpu.VMEM((PER_W,), jnp.int32),
            pltpu.VMEM((NBUF, BUF_ROWS * EMBED_DIM), jnp.float32),
            pltpu.VMEM((VOCAB_ROWS * EMBED_DIM,), jnp.float32),
        ]
        + [pltpu.SemaphoreType.DMA] * NBUF,
    )
    def k(idx_hbm, table_hbm, out_hbm, idx_v, rows, table_v,
          o0, o1, o2, o3):
        osem = [o0, o1, o2, o3]
        wid = lax.axis_index("s") * _NC + lax.axis_index("c")
        base = wid * PER_W
        obase = base * EMBED_DIM

        pltpu.sync_copy(table_hbm, table_v)
        pltpu.sync_copy(idx_hbm.at[pl.ds(base, PER_W)], idx_v)

        def compute(s, b):
            buf = rows.at[b]

            def group(g, carry):
                idx_vec = idx_v[pl.ds(s * BUF_ROWS + g * _L, _L)]
                gvec = idx_vec << 6
                for j in range(_L):
                    src = gvec[j]
                    dst = g * (_L * EMBED_DIM) + j * EMBED_DIM
                    for t in range(EMBED_DIM // _L):
                        buf[pl.ds(dst + t * _L, _L)] = (
                            table_v[pl.ds(src + t * _L, _L)])
                return carry

            lax.fori_loop(0, NGROUP, group, 0)

        def write(s, b, start):
            cp = pltpu.make_async_copy(
                rows.at[b],
                out_hbm.at[pl.ds(obase + s * (BUF_ROWS * EMBED_DIM),
                                 BUF_ROWS * EMBED_DIM)],
                osem[b],
            )
            cp.start() if start else cp.wait()

        for b in range(NBUF):
            compute(b, b)
            write(b, b, True)

        def body(i, carry):
            sbase = i * NBUF
            for b in range(NBUF):
                s = sbase + b
                write(s - NBUF, b, False)
                compute(s, b)
                write(s, b, True)
            return carry

        lax.fori_loop(1, NSTEP // NBUF, body, 0)

        for b in range(NBUF):
            write(NSTEP - NBUF + b, b, False)

    return k


_sc_gather = _make_kernel()


def kernel(inputs, embedding_table):
    idx = inputs.reshape(TOTAL)
    table = embedding_table.reshape(VOCAB_ROWS * EMBED_DIM)
    out = _sc_gather(idx, table)
    return out.reshape(BATCH, SEQ, EMBED_DIM)
